# baseline (speedup 1.0000x reference)
import jax
import jax.numpy as jnp
from jax.experimental import pallas as pl

B, C, N, S, D1, D2 = 4, 3, 8192, 2048, 32, 64
MLP0, MLP1 = 128, 128

TN1 = 256
NT1 = N // TN1
TN2 = 1024
TN3 = 2048


def _knn_layer1_kernel(x1_ref, x1sq_ref, x2_ref, p2_ref, p1_ref, w0t_ref,
                       b0_ref, z_ref, s_ref, sq_ref):
    x1 = x1_ref[0]
    x2 = x2_ref[0]
    x1sq = x1sq_ref[0]
    x2sq = jnp.sum(x2 * x2, axis=0, keepdims=True)
    d = (x1sq + x2sq
         - 2.0 * jnp.dot(x1, x2, preferred_element_type=jnp.float32))

    iota = jax.lax.broadcasted_iota(jnp.int32, (TN1, S), 1)
    sels, recips = [], []
    for _ in range(3):
        v = jnp.min(d, axis=1, keepdims=True)
        idx = jnp.min(jnp.where(d == v, iota, S), axis=1, keepdims=True)
        sel = iota == idx
        sels.append(sel)
        recips.append(1.0 / (v + 1e-8))
        d = jnp.where(sel, jnp.float32(jnp.inf), d)
    norm = recips[0] + recips[1] + recips[2]
    a = jnp.zeros((TN1, S), jnp.float32)
    for k in range(3):
        a = jnp.where(sels[k], recips[k] / norm, a)

    interp = jnp.dot(a, p2_ref[0], preferred_element_type=jnp.float32,
                     precision=jax.lax.Precision.HIGHEST)
    p = jnp.concatenate([interp, p1_ref[0]], axis=1)
    z = jnp.dot(p, w0t_ref[...], preferred_element_type=jnp.float32) + b0_ref[...]
    z_ref[0] = z

    @pl.when((pl.program_id(0) == 0) & (pl.program_id(1) == 0))
    def _init():
        s_ref[...] = jnp.zeros_like(s_ref)
        sq_ref[...] = jnp.zeros_like(sq_ref)

    s_ref[...] += jnp.sum(z, axis=0, keepdims=True)
    sq_ref[...] += jnp.sum(z * z, axis=0, keepdims=True)


def _layer2_kernel(z_ref, sc_ref, sh_ref, w1t_ref, b1_ref,
                   z2_ref, s_ref, sq_ref):
    h = jnp.maximum(z_ref[...] * sc_ref[...] + sh_ref[...], 0.0)
    z2 = jnp.dot(h, w1t_ref[...], preferred_element_type=jnp.float32) + b1_ref[...]
    z2_ref[...] = z2

    @pl.when(pl.program_id(0) == 0)
    def _init():
        s_ref[...] = jnp.zeros_like(s_ref)
        sq_ref[...] = jnp.zeros_like(sq_ref)

    s_ref[...] += jnp.sum(z2, axis=0, keepdims=True)
    sq_ref[...] += jnp.sum(z2 * z2, axis=0, keepdims=True)


def _final_kernel(z2_ref, sc_ref, sh_ref, out_ref):
    y = jnp.maximum(z2_ref[0] * sc_ref[...] + sh_ref[...], 0.0)
    out_ref[0] = y.T


def kernel(xyz1, xyz2, points1, points2, W0, b0, g0, beta0, W1, b1, g1, beta1):
    f32 = jnp.float32
    xyz1_t = jnp.transpose(xyz1, (0, 2, 1))
    x1sq = jnp.sum(xyz1_t ** 2, axis=-1)[..., None]
    p2_t = jnp.transpose(points2, (0, 2, 1))
    p1_t = jnp.transpose(points1, (0, 2, 1))
    w0t = W0.T
    w1t = W1.T
    b0r = b0.reshape(1, MLP0)
    b1r = b1.reshape(1, MLP1)

    z1, s1, sq1 = pl.pallas_call(
        _knn_layer1_kernel,
        grid=(B, NT1),
        in_specs=[
            pl.BlockSpec((1, TN1, C), lambda b, n: (b, n, 0)),
            pl.BlockSpec((1, TN1, 1), lambda b, n: (b, n, 0)),
            pl.BlockSpec((1, C, S), lambda b, n: (b, 0, 0)),
            pl.BlockSpec((1, S, D2), lambda b, n: (b, 0, 0)),
            pl.BlockSpec((1, TN1, D1), lambda b, n: (b, n, 0)),
            pl.BlockSpec((D1 + D2, MLP0), lambda b, n: (0, 0)),
            pl.BlockSpec((1, MLP0), lambda b, n: (0, 0)),
        ],
        out_specs=[
            pl.BlockSpec((1, TN1, MLP0), lambda b, n: (b, n, 0)),
            pl.BlockSpec((1, MLP0), lambda b, n: (0, 0)),
            pl.BlockSpec((1, MLP0), lambda b, n: (0, 0)),
        ],
        out_shape=[
            jax.ShapeDtypeStruct((B, N, MLP0), f32),
            jax.ShapeDtypeStruct((1, MLP0), f32),
            jax.ShapeDtypeStruct((1, MLP0), f32),
        ],
    )(xyz1_t, x1sq, xyz2, p2_t, p1_t, w0t, b0r)

    cnt = f32(B * N)
    mean0 = s1 / cnt
    var0 = sq1 / cnt - mean0 * mean0
    sc0 = (g0.reshape(1, MLP0) / jnp.sqrt(var0 + 1e-5)).astype(f32)
    sh0 = beta0.reshape(1, MLP0) - mean0 * sc0

    z1f = z1.reshape(B * N, MLP0)
    z2, s2, sq2 = pl.pallas_call(
        _layer2_kernel,
        grid=(B * N // TN2,),
        in_specs=[
            pl.BlockSpec((TN2, MLP0), lambda i: (i, 0)),
            pl.BlockSpec((1, MLP0), lambda i: (0, 0)),
            pl.BlockSpec((1, MLP0), lambda i: (0, 0)),
            pl.BlockSpec((MLP0, MLP1), lambda i: (0, 0)),
            pl.BlockSpec((1, MLP1), lambda i: (0, 0)),
        ],
        out_specs=[
            pl.BlockSpec((TN2, MLP1), lambda i: (i, 0)),
            pl.BlockSpec((1, MLP1), lambda i: (0, 0)),
            pl.BlockSpec((1, MLP1), lambda i: (0, 0)),
        ],
        out_shape=[
            jax.ShapeDtypeStruct((B * N, MLP1), f32),
            jax.ShapeDtypeStruct((1, MLP1), f32),
            jax.ShapeDtypeStruct((1, MLP1), f32),
        ],
    )(z1f, sc0, sh0, w1t, b1r)

    mean1 = s2 / cnt
    var1 = sq2 / cnt - mean1 * mean1
    sc1 = (g1.reshape(1, MLP1) / jnp.sqrt(var1 + 1e-5)).astype(f32)
    sh1 = beta1.reshape(1, MLP1) - mean1 * sc1

    z2r = z2.reshape(B, N, MLP1)
    out = pl.pallas_call(
        _final_kernel,
        grid=(B, N // TN3),
        in_specs=[
            pl.BlockSpec((1, TN3, MLP1), lambda b, n: (b, n, 0)),
            pl.BlockSpec((1, MLP1), lambda b, n: (0, 0)),
            pl.BlockSpec((1, MLP1), lambda b, n: (0, 0)),
        ],
        out_specs=pl.BlockSpec((1, MLP1, TN3), lambda b, n: (b, 0, n)),
        out_shape=jax.ShapeDtypeStruct((B, MLP1, N), f32),
    )(z2r, sc1, sh1)

    return out


# speedup vs baseline: 1.5094x; 1.5094x over previous
import jax
import jax.numpy as jnp
from jax.experimental import pallas as pl

B, C, N, S, D1, D2 = 4, 3, 8192, 2048, 32, 64
MLP0, MLP1 = 128, 128

TN1 = 256
NT1 = N // TN1
TN2 = 1024
TN3 = 2048


def _knn_layer1_kernel(x1_ref, x1sq_ref, x2_ref, p2_ref, p1_ref, w0t_ref,
                       b0_ref, z_ref, s_ref, sq_ref):
    x1 = x1_ref[0]
    x2 = x2_ref[0]
    x1sq = x1sq_ref[0]
    x2sq = jnp.sum(x2 * x2, axis=0, keepdims=True)
    d = (x1sq + x2sq
         - 2.0 * jnp.dot(x1, x2, preferred_element_type=jnp.float32))

    iota = jax.lax.broadcasted_iota(jnp.int32, (TN1, S), 1)
    sels, recips = [], []
    for _ in range(3):
        v = jnp.min(d, axis=1, keepdims=True)
        idx = jnp.argmin(d, axis=1)[:, None]
        sel = iota == idx
        sels.append(sel)
        recips.append(1.0 / (v + 1e-8))
        d = jnp.where(sel, jnp.float32(jnp.inf), d)
    norm = recips[0] + recips[1] + recips[2]
    a = jnp.zeros((TN1, S), jnp.float32)
    for k in range(3):
        a = jnp.where(sels[k], recips[k] / norm, a)

    interp = jnp.dot(a, p2_ref[0], preferred_element_type=jnp.float32)
    p = jnp.concatenate([interp, p1_ref[0]], axis=1)
    z = jnp.dot(p, w0t_ref[...], preferred_element_type=jnp.float32) + b0_ref[...]
    z_ref[0] = z

    @pl.when((pl.program_id(0) == 0) & (pl.program_id(1) == 0))
    def _init():
        s_ref[...] = jnp.zeros_like(s_ref)
        sq_ref[...] = jnp.zeros_like(sq_ref)

    s_ref[...] += jnp.sum(z, axis=0, keepdims=True)
    sq_ref[...] += jnp.sum(z * z, axis=0, keepdims=True)


def _layer2_kernel(z_ref, sc_ref, sh_ref, w1t_ref, b1_ref,
                   z2_ref, s_ref, sq_ref):
    h = jnp.maximum(z_ref[...] * sc_ref[...] + sh_ref[...], 0.0)
    z2 = jnp.dot(h, w1t_ref[...], preferred_element_type=jnp.float32) + b1_ref[...]
    z2_ref[...] = z2

    @pl.when(pl.program_id(0) == 0)
    def _init():
        s_ref[...] = jnp.zeros_like(s_ref)
        sq_ref[...] = jnp.zeros_like(sq_ref)

    s_ref[...] += jnp.sum(z2, axis=0, keepdims=True)
    sq_ref[...] += jnp.sum(z2 * z2, axis=0, keepdims=True)


def _final_kernel(z2_ref, sc_ref, sh_ref, out_ref):
    y = jnp.maximum(z2_ref[0] * sc_ref[...] + sh_ref[...], 0.0)
    out_ref[0] = y.T


def kernel(xyz1, xyz2, points1, points2, W0, b0, g0, beta0, W1, b1, g1, beta1):
    f32 = jnp.float32
    xyz1_t = jnp.transpose(xyz1, (0, 2, 1))
    x1sq = jnp.sum(xyz1_t ** 2, axis=-1)[..., None]
    p2_t = jnp.transpose(points2, (0, 2, 1))
    p1_t = jnp.transpose(points1, (0, 2, 1))
    w0t = W0.T
    w1t = W1.T
    b0r = b0.reshape(1, MLP0)
    b1r = b1.reshape(1, MLP1)

    z1, s1, sq1 = pl.pallas_call(
        _knn_layer1_kernel,
        grid=(B, NT1),
        in_specs=[
            pl.BlockSpec((1, TN1, C), lambda b, n: (b, n, 0)),
            pl.BlockSpec((1, TN1, 1), lambda b, n: (b, n, 0)),
            pl.BlockSpec((1, C, S), lambda b, n: (b, 0, 0)),
            pl.BlockSpec((1, S, D2), lambda b, n: (b, 0, 0)),
            pl.BlockSpec((1, TN1, D1), lambda b, n: (b, n, 0)),
            pl.BlockSpec((D1 + D2, MLP0), lambda b, n: (0, 0)),
            pl.BlockSpec((1, MLP0), lambda b, n: (0, 0)),
        ],
        out_specs=[
            pl.BlockSpec((1, TN1, MLP0), lambda b, n: (b, n, 0)),
            pl.BlockSpec((1, MLP0), lambda b, n: (0, 0)),
            pl.BlockSpec((1, MLP0), lambda b, n: (0, 0)),
        ],
        out_shape=[
            jax.ShapeDtypeStruct((B, N, MLP0), f32),
            jax.ShapeDtypeStruct((1, MLP0), f32),
            jax.ShapeDtypeStruct((1, MLP0), f32),
        ],
    )(xyz1_t, x1sq, xyz2, p2_t, p1_t, w0t, b0r)

    cnt = f32(B * N)
    mean0 = s1 / cnt
    var0 = sq1 / cnt - mean0 * mean0
    sc0 = (g0.reshape(1, MLP0) / jnp.sqrt(var0 + 1e-5)).astype(f32)
    sh0 = beta0.reshape(1, MLP0) - mean0 * sc0

    z1f = z1.reshape(B * N, MLP0)
    z2, s2, sq2 = pl.pallas_call(
        _layer2_kernel,
        grid=(B * N // TN2,),
        in_specs=[
            pl.BlockSpec((TN2, MLP0), lambda i: (i, 0)),
            pl.BlockSpec((1, MLP0), lambda i: (0, 0)),
            pl.BlockSpec((1, MLP0), lambda i: (0, 0)),
            pl.BlockSpec((MLP0, MLP1), lambda i: (0, 0)),
            pl.BlockSpec((1, MLP1), lambda i: (0, 0)),
        ],
        out_specs=[
            pl.BlockSpec((TN2, MLP1), lambda i: (i, 0)),
            pl.BlockSpec((1, MLP1), lambda i: (0, 0)),
            pl.BlockSpec((1, MLP1), lambda i: (0, 0)),
        ],
        out_shape=[
            jax.ShapeDtypeStruct((B * N, MLP1), f32),
            jax.ShapeDtypeStruct((1, MLP1), f32),
            jax.ShapeDtypeStruct((1, MLP1), f32),
        ],
    )(z1f, sc0, sh0, w1t, b1r)

    mean1 = s2 / cnt
    var1 = sq2 / cnt - mean1 * mean1
    sc1 = (g1.reshape(1, MLP1) / jnp.sqrt(var1 + 1e-5)).astype(f32)
    sh1 = beta1.reshape(1, MLP1) - mean1 * sc1

    z2r = z2.reshape(B, N, MLP1)
    out = pl.pallas_call(
        _final_kernel,
        grid=(B, N // TN3),
        in_specs=[
            pl.BlockSpec((1, TN3, MLP1), lambda b, n: (b, n, 0)),
            pl.BlockSpec((1, MLP1), lambda b, n: (0, 0)),
            pl.BlockSpec((1, MLP1), lambda b, n: (0, 0)),
        ],
        out_specs=pl.BlockSpec((1, MLP1, TN3), lambda b, n: (b, 0, n)),
        out_shape=jax.ShapeDtypeStruct((B, MLP1, N), f32),
    )(z2r, sc1, sh1)

    return out


# speedup vs baseline: 1.5203x; 1.0072x over previous
import jax
import jax.numpy as jnp
from jax.experimental import pallas as pl

B, C, N, S, D1, D2 = 4, 3, 8192, 2048, 32, 64
MLP0, MLP1 = 128, 128

TN1 = 512
NT1 = N // TN1
TN2 = 1024
TN3 = 2048


def _knn_layer1_kernel(x1_ref, x1sq_ref, x2_ref, p2_ref, p1_ref, w0t_ref,
                       b0_ref, z_ref, s_ref, sq_ref):
    x1 = x1_ref[0]
    x2 = x2_ref[0]
    x1sq = x1sq_ref[0]
    x2sq = jnp.sum(x2 * x2, axis=0, keepdims=True)
    d = (x1sq + x2sq
         - 2.0 * jnp.dot(x1, x2, preferred_element_type=jnp.float32))

    iota = jax.lax.broadcasted_iota(jnp.int32, (TN1, S), 1)
    sels, recips = [], []
    for r in range(3):
        v = jnp.min(d, axis=1, keepdims=True)
        idx = jnp.argmin(d, axis=1)[:, None]
        sel = iota == idx
        sels.append(sel)
        recips.append(1.0 / (v + 1e-8))
        if r < 2:
            d = jnp.where(sel, jnp.float32(jnp.inf), d)
    norm = recips[0] + recips[1] + recips[2]
    a = jnp.zeros((TN1, S), jnp.float32)
    for k in range(3):
        a = jnp.where(sels[k], recips[k] / norm, a)

    interp = jnp.dot(a, p2_ref[0], preferred_element_type=jnp.float32)
    p = jnp.concatenate([interp, p1_ref[0]], axis=1)
    z = jnp.dot(p, w0t_ref[...], preferred_element_type=jnp.float32) + b0_ref[...]
    z_ref[0] = z

    @pl.when((pl.program_id(0) == 0) & (pl.program_id(1) == 0))
    def _init():
        s_ref[...] = jnp.zeros_like(s_ref)
        sq_ref[...] = jnp.zeros_like(sq_ref)

    s_ref[...] += jnp.sum(z, axis=0, keepdims=True)
    sq_ref[...] += jnp.sum(z * z, axis=0, keepdims=True)


def _layer2_kernel(z_ref, sc_ref, sh_ref, w1t_ref, b1_ref,
                   z2_ref, s_ref, sq_ref):
    h = jnp.maximum(z_ref[...] * sc_ref[...] + sh_ref[...], 0.0)
    z2 = jnp.dot(h, w1t_ref[...], preferred_element_type=jnp.float32) + b1_ref[...]
    z2_ref[...] = z2

    @pl.when(pl.program_id(0) == 0)
    def _init():
        s_ref[...] = jnp.zeros_like(s_ref)
        sq_ref[...] = jnp.zeros_like(sq_ref)

    s_ref[...] += jnp.sum(z2, axis=0, keepdims=True)
    sq_ref[...] += jnp.sum(z2 * z2, axis=0, keepdims=True)


def _final_kernel(z2_ref, sc_ref, sh_ref, out_ref):
    y = jnp.maximum(z2_ref[0] * sc_ref[...] + sh_ref[...], 0.0)
    out_ref[0] = y.T


def kernel(xyz1, xyz2, points1, points2, W0, b0, g0, beta0, W1, b1, g1, beta1):
    f32 = jnp.float32
    xyz1_t = jnp.transpose(xyz1, (0, 2, 1))
    x1sq = jnp.sum(xyz1_t ** 2, axis=-1)[..., None]
    p2_t = jnp.transpose(points2, (0, 2, 1))
    p1_t = jnp.transpose(points1, (0, 2, 1))
    w0t = W0.T
    w1t = W1.T
    b0r = b0.reshape(1, MLP0)
    b1r = b1.reshape(1, MLP1)

    z1, s1, sq1 = pl.pallas_call(
        _knn_layer1_kernel,
        grid=(B, NT1),
        in_specs=[
            pl.BlockSpec((1, TN1, C), lambda b, n: (b, n, 0)),
            pl.BlockSpec((1, TN1, 1), lambda b, n: (b, n, 0)),
            pl.BlockSpec((1, C, S), lambda b, n: (b, 0, 0)),
            pl.BlockSpec((1, S, D2), lambda b, n: (b, 0, 0)),
            pl.BlockSpec((1, TN1, D1), lambda b, n: (b, n, 0)),
            pl.BlockSpec((D1 + D2, MLP0), lambda b, n: (0, 0)),
            pl.BlockSpec((1, MLP0), lambda b, n: (0, 0)),
        ],
        out_specs=[
            pl.BlockSpec((1, TN1, MLP0), lambda b, n: (b, n, 0)),
            pl.BlockSpec((1, MLP0), lambda b, n: (0, 0)),
            pl.BlockSpec((1, MLP0), lambda b, n: (0, 0)),
        ],
        out_shape=[
            jax.ShapeDtypeStruct((B, N, MLP0), f32),
            jax.ShapeDtypeStruct((1, MLP0), f32),
            jax.ShapeDtypeStruct((1, MLP0), f32),
        ],
    )(xyz1_t, x1sq, xyz2, p2_t, p1_t, w0t, b0r)

    cnt = f32(B * N)
    mean0 = s1 / cnt
    var0 = sq1 / cnt - mean0 * mean0
    sc0 = (g0.reshape(1, MLP0) / jnp.sqrt(var0 + 1e-5)).astype(f32)
    sh0 = beta0.reshape(1, MLP0) - mean0 * sc0

    z1f = z1.reshape(B * N, MLP0)
    z2, s2, sq2 = pl.pallas_call(
        _layer2_kernel,
        grid=(B * N // TN2,),
        in_specs=[
            pl.BlockSpec((TN2, MLP0), lambda i: (i, 0)),
            pl.BlockSpec((1, MLP0), lambda i: (0, 0)),
            pl.BlockSpec((1, MLP0), lambda i: (0, 0)),
            pl.BlockSpec((MLP0, MLP1), lambda i: (0, 0)),
            pl.BlockSpec((1, MLP1), lambda i: (0, 0)),
        ],
        out_specs=[
            pl.BlockSpec((TN2, MLP1), lambda i: (i, 0)),
            pl.BlockSpec((1, MLP1), lambda i: (0, 0)),
            pl.BlockSpec((1, MLP1), lambda i: (0, 0)),
        ],
        out_shape=[
            jax.ShapeDtypeStruct((B * N, MLP1), f32),
            jax.ShapeDtypeStruct((1, MLP1), f32),
            jax.ShapeDtypeStruct((1, MLP1), f32),
        ],
    )(z1f, sc0, sh0, w1t, b1r)

    mean1 = s2 / cnt
    var1 = sq2 / cnt - mean1 * mean1
    sc1 = (g1.reshape(1, MLP1) / jnp.sqrt(var1 + 1e-5)).astype(f32)
    sh1 = beta1.reshape(1, MLP1) - mean1 * sc1

    z2r = z2.reshape(B, N, MLP1)
    out = pl.pallas_call(
        _final_kernel,
        grid=(B, N // TN3),
        in_specs=[
            pl.BlockSpec((1, TN3, MLP1), lambda b, n: (b, n, 0)),
            pl.BlockSpec((1, MLP1), lambda b, n: (0, 0)),
            pl.BlockSpec((1, MLP1), lambda b, n: (0, 0)),
        ],
        out_specs=pl.BlockSpec((1, MLP1, TN3), lambda b, n: (b, 0, n)),
        out_shape=jax.ShapeDtypeStruct((B, MLP1, N), f32),
    )(z2r, sc1, sh1)

    return out
